# Initial kernel scaffold; baseline (speedup 1.0000x reference)
import jax
import jax.numpy as jnp
from jax.experimental import pallas as pl


def kernel(x, mask, embed_table, W1, b1, W2, b2):
    raise NotImplementedError("write your pallas kernel here")


# speedup vs baseline: 4.7895x; 4.7895x over previous
import functools

import jax
import jax.numpy as jnp
from jax import lax
from jax.experimental import pallas as pl
from jax.experimental.pallas import tpu as pltpu
from jax.experimental.pallas import tpu_sc as plsc

B = 16384
L = 200
D = 50
DP = 64
H = 100
O = 2

_info = plsc.get_sparse_core_info()
NC, NS = _info.num_cores, _info.num_subcores
NW = NC * NS
BPW = B // NW


def _sc_pool(x, mask, tblp):
    mesh = plsc.VectorSubcoreMesh(core_axis_name="c", subcore_axis_name="s")

    LP = 208

    @functools.partial(
        pl.kernel,
        mesh=mesh,
        compiler_params=pltpu.CompilerParams(use_tc_tiling_on_sc=False),
        out_type=jax.ShapeDtypeStruct((B * DP,), jnp.float32),
        scratch_types=[
            pltpu.VMEM((LP,), jnp.int32),
            pltpu.VMEM((LP,), jnp.float32),
            pltpu.VMEM((LP, DP), jnp.float32),
            pltpu.VMEM((DP,), jnp.float32),
            pltpu.SemaphoreType.DMA,
        ],
    )
    def k(x_hbm, m_hbm, t_hbm, out_hbm, idx_v, msk_v, rows_v, ob, sem):
        wid = lax.axis_index("s") * NC + lax.axis_index("c")
        base = wid * BPW
        zf = jnp.zeros((16,), jnp.float32)
        zi = jnp.zeros((16,), jnp.int32)
        idx_v[pl.ds(LP - 16, 16)] = zi
        msk_v[pl.ds(LP - 16, 16)] = zf

        def row(i, carry):
            b = base + i
            pltpu.sync_copy(x_hbm.at[pl.ds(b * L, L)], idx_v.at[pl.ds(0, L)])
            pltpu.sync_copy(m_hbm.at[pl.ds(b * L, L)], msk_v.at[pl.ds(0, L)])
            pltpu.async_copy(t_hbm.at[idx_v], rows_v, sem).wait()

            def gbody(g, accs):
                mv = msk_v[pl.ds(g * 16, 16)]
                for j in range(16):
                    lidx = g * 16 + j
                    bm = jnp.full((16,), mv[j], dtype=jnp.float32)
                    accs = tuple(
                        accs[c] + bm * rows_v[lidx, pl.ds(c * 16, 16)]
                        for c in range(4)
                    )
                return accs

            accs = lax.fori_loop(0, LP // 16, gbody, (zf, zf, zf, zf))
            for c in range(4):
                ob[pl.ds(c * 16, 16)] = accs[c]
            pltpu.sync_copy(ob, out_hbm.at[pl.ds(b * DP, DP)])
            return carry

        lax.fori_loop(0, BPW, row, 0)

    return k(x.reshape(B * L), mask.reshape(B * L), tblp).reshape(B, DP)


def _mlp(pooled, w1p, b1, w2, b2):
    TB = 128

    def body(p_ref, w1_ref, b1_ref, w2_ref, b2_ref, o_ref):
        p = p_ref[...]
        h = jnp.dot(p, w1_ref[...], preferred_element_type=jnp.float32)
        h = jnp.maximum(h + b1_ref[...], 0.0)
        o_ref[...] = (
            jnp.dot(h, w2_ref[...], preferred_element_type=jnp.float32)
            + b2_ref[...]
        )

    return pl.pallas_call(
        body,
        grid=(B // TB,),
        in_specs=[
            pl.BlockSpec((TB, DP), lambda i: (i, 0)),
            pl.BlockSpec((DP, H), lambda i: (0, 0)),
            pl.BlockSpec((1, H), lambda i: (0, 0)),
            pl.BlockSpec((H, O), lambda i: (0, 0)),
            pl.BlockSpec((1, O), lambda i: (0, 0)),
        ],
        out_specs=pl.BlockSpec((TB, O), lambda i: (i, 0)),
        out_shape=jax.ShapeDtypeStruct((B, O), jnp.float32),
    )(pooled, w1p, b1, w2, b2)


def kernel(x, mask, embed_table, W1, b1, W2, b2):
    tblp = jnp.pad(embed_table, ((0, 0), (0, DP - D)))
    w1p = jnp.pad(W1 * (1.0 / L), ((0, DP - D), (0, 0)))
    pooled = _sc_pool(x, mask, tblp)
    return _mlp(pooled, w1p, b1.reshape(1, H), W2, b2.reshape(1, O))


# speedup vs baseline: 26.8300x; 5.6019x over previous
import functools

import jax
import jax.numpy as jnp
from jax import lax
from jax.experimental import pallas as pl
from jax.experimental.pallas import tpu as pltpu
from jax.experimental.pallas import tpu_sc as plsc

B = 16384
L = 200
D = 50
DP = 64
H = 100
O = 2

_info = plsc.get_sparse_core_info()
NC, NS = _info.num_cores, _info.num_subcores
NW = NC * NS
BPW = B // NW
CH = 4
CHL = CH * L
NCHUNK = BPW // CH


def _sc_pool(x, mask, tblp):
    mesh = plsc.VectorSubcoreMesh(core_axis_name="c", subcore_axis_name="s")

    @functools.partial(
        pl.kernel,
        mesh=mesh,
        compiler_params=pltpu.CompilerParams(use_tc_tiling_on_sc=False),
        out_type=jax.ShapeDtypeStruct((B * DP,), jnp.float32),
        scratch_types=[
            pltpu.VMEM((2, CHL), jnp.int32),
            pltpu.VMEM((2, CHL), jnp.float32),
            pltpu.VMEM((2, CHL, DP), jnp.float32),
            pltpu.VMEM((2, CH * DP), jnp.float32),
            pltpu.SemaphoreType.DMA((2,)),
            pltpu.SemaphoreType.DMA((2,)),
            pltpu.SemaphoreType.DMA((2,)),
            pltpu.SemaphoreType.DMA((2,)),
        ],
    )
    def k(x_hbm, m_hbm, t_hbm, out_hbm, idxb, mskb, rowsb, ob, si, sm, sr, so):
        wid = lax.axis_index("s") * NC + lax.axis_index("c")
        base = wid * BPW

        def cp_idx(c, s):
            off = (base + c * CH) * L
            pltpu.async_copy(x_hbm.at[pl.ds(off, CHL)], idxb.at[s], si.at[s])

        def cp_msk(c, s):
            off = (base + c * CH) * L
            pltpu.async_copy(m_hbm.at[pl.ds(off, CHL)], mskb.at[s], sm.at[s])

        def wait_idx(s):
            pltpu.make_async_copy(
                x_hbm.at[pl.ds(0, CHL)], idxb.at[s], si.at[s]).wait()

        def wait_msk(s):
            pltpu.make_async_copy(
                m_hbm.at[pl.ds(0, CHL)], mskb.at[s], sm.at[s]).wait()

        def gather(s):
            pltpu.async_copy(t_hbm.at[idxb.at[s]], rowsb.at[s], sr.at[s])

        def wait_gather(s):
            pltpu.make_async_copy(
                t_hbm.at[idxb.at[0]], rowsb.at[s], sr.at[s]).wait()

        def out_copy(c, s):
            off = (base + c * CH) * DP
            pltpu.async_copy(
                ob.at[s], out_hbm.at[pl.ds(off, CH * DP)], so.at[s])

        def wait_out(s):
            pltpu.make_async_copy(
                ob.at[s], out_hbm.at[pl.ds(0, CH * DP)], so.at[s]).wait()

        def compute(s):
            for r in range(CH):
                rbase = r * L

                def gbody(g, accs, rbase=rbase):
                    mv = mskb[s, pl.ds(rbase + g * 16, 16)]
                    for j in range(16):
                        lidx = rbase + g * 16 + j
                        bm = jnp.full((16,), mv[j], dtype=jnp.float32)
                        accs = tuple(
                            accs[cc] + bm * rowsb[s, lidx, pl.ds(cc * 16, 16)]
                            for cc in range(4)
                        )
                    return accs

                zf = jnp.zeros((16,), jnp.float32)
                accs = lax.fori_loop(0, (L // 16), gbody, (zf, zf, zf, zf))
                mv = mskb[s, pl.ds(rbase + L - 16, 16)]
                for j in range(8, 16):
                    lidx = rbase + L - 16 + j
                    bm = jnp.full((16,), mv[j], dtype=jnp.float32)
                    accs = tuple(
                        accs[cc] + bm * rowsb[s, lidx, pl.ds(cc * 16, 16)]
                        for cc in range(4)
                    )
                for cc in range(4):
                    ob[s, pl.ds(r * DP + cc * 16, 16)] = accs[cc]

        cp_idx(0, 0)
        cp_msk(0, 0)
        wait_idx(0)
        gather(0)
        cp_idx(1, 1)
        cp_msk(1, 1)

        def body(c, carry):
            s = c % 2
            wait_gather(s)

            @pl.when(c + 2 < NCHUNK)
            def _():
                cp_idx(c + 2, s)

            @pl.when(c + 1 < NCHUNK)
            def _():
                wait_idx(1 - s)
                gather(1 - s)

            wait_msk(s)

            @pl.when(c >= 2)
            def _():
                wait_out(s)

            compute(s)

            @pl.when(c + 2 < NCHUNK)
            def _():
                cp_msk(c + 2, s)

            out_copy(c, s)
            return carry

        lax.fori_loop(0, NCHUNK, body, 0)
        wait_out(0)
        wait_out(1)

    return k(x.reshape(B * L), mask.reshape(B * L), tblp).reshape(B, DP)


def _mlp(pooled, w1p, b1, w2, b2):
    TB = 512

    def body(p_ref, w1_ref, b1_ref, w2_ref, b2_ref, o_ref):
        p = p_ref[...]
        h = jnp.dot(p, w1_ref[...], preferred_element_type=jnp.float32)
        h = jnp.maximum(h + b1_ref[...], 0.0)
        o_ref[...] = (
            jnp.dot(h, w2_ref[...], preferred_element_type=jnp.float32)
            + b2_ref[...]
        )

    return pl.pallas_call(
        body,
        grid=(B // TB,),
        in_specs=[
            pl.BlockSpec((TB, DP), lambda i: (i, 0)),
            pl.BlockSpec((DP, H), lambda i: (0, 0)),
            pl.BlockSpec((1, H), lambda i: (0, 0)),
            pl.BlockSpec((H, O), lambda i: (0, 0)),
            pl.BlockSpec((1, O), lambda i: (0, 0)),
        ],
        out_specs=pl.BlockSpec((TB, O), lambda i: (i, 0)),
        out_shape=jax.ShapeDtypeStruct((B, O), jnp.float32),
    )(pooled, w1p, b1, w2, b2)


def kernel(x, mask, embed_table, W1, b1, W2, b2):
    tblp = jnp.pad(embed_table, ((0, 0), (0, DP - D)))
    w1p = jnp.pad(W1 * (1.0 / L), ((0, DP - D), (0, 0)))
    pooled = _sc_pool(x, mask, tblp)
    return _mlp(pooled, w1p, b1.reshape(1, H), W2, b2.reshape(1, O))
